# baseline (speedup 1.0000x reference)
import functools

import jax
import jax.numpy as jnp
from jax import lax
from jax.experimental import pallas as pl
from jax.experimental.pallas import tpu as pltpu
from jax.experimental.pallas import tpu_sc as plsc

VOCAB = 8192
D = 8192
B = 8192
B_TC = 2048
B_SC = B - B_TC

NC = 2
NS = 16
NW = NC * NS
BPW = B_SC // NW
CHUNK = 4
NCHUNK = BPW // CHUNK
NBUF = 3

RPS = 16


@jax.jit
def _sc_gather(idx, table):
    mesh = plsc.VectorSubcoreMesh(core_axis_name="c", subcore_axis_name="s")

    @functools.partial(
        pl.kernel,
        out_type=jax.ShapeDtypeStruct((B_SC, D), jnp.float32),
        mesh=mesh,
        scratch_types=[
            pltpu.VMEM((NCHUNK, CHUNK), jnp.int32),
            pltpu.VMEM((NBUF, CHUNK, D), jnp.float32),
            pltpu.SemaphoreType.DMA,
            pltpu.SemaphoreType.DMA,
            pltpu.SemaphoreType.DMA,
            pltpu.SemaphoreType.DMA,
            pltpu.SemaphoreType.DMA,
            pltpu.SemaphoreType.DMA,
        ],
    )
    def k(idx_hbm, table_hbm, out_hbm, idx_v, bufs, g0, g1, g2, s0, s1, s2):
        wid = lax.axis_index("s") * NC + lax.axis_index("c")
        base = wid * BPW
        pltpu.sync_copy(idx_hbm.at[wid], idx_v)

        gsems = (g0, g1, g2)
        ssems = (s0, s1, s2)

        def g_start(c, b):
            pltpu.async_copy(table_hbm.at[idx_v.at[c]], bufs.at[b], gsems[b])

        def g_wait(b):
            pltpu.make_async_copy(
                table_hbm.at[pl.ds(0, CHUNK)], bufs.at[b], gsems[b]
            ).wait()

        def s_start(c, b):
            pltpu.async_copy(
                bufs.at[b], out_hbm.at[pl.ds(base + c * CHUNK, CHUNK)], ssems[b]
            )

        def s_wait(b):
            pltpu.make_async_copy(
                bufs.at[b], out_hbm.at[pl.ds(base, CHUNK)], ssems[b]
            ).wait()

        g_start(0, 0)
        g_start(1, 1)
        for c in range(NCHUNK):
            b = c % NBUF
            g_wait(b)
            s_start(c, b)
            if c + 2 <= NCHUNK - 1:
                bn = (c + 2) % NBUF
                if c != 0:
                    s_wait(bn)
                g_start(c + 2, bn)
        s_wait((NCHUNK - 2) % NBUF)
        s_wait((NCHUNK - 1) % NBUF)
        s_wait((NCHUNK - 3) % NBUF)

    return k(idx, table)


def _tc_body(idx_ref, *refs):
    ins = refs[:RPS]
    out_ref = refs[RPS]
    for j in range(RPS):
        out_ref[j, :] = ins[j][0, 0, :]


@jax.jit
def _tc_gather(idx, table3):
    grid = (B_TC // RPS,)
    in_specs = [
        pl.BlockSpec((1, 1, D), (lambda i, idx_ref, j=j: (idx_ref[RPS * i + j], 0, 0)))
        for j in range(RPS)
    ]
    return pl.pallas_call(
        _tc_body,
        grid_spec=pltpu.PrefetchScalarGridSpec(
            num_scalar_prefetch=1,
            grid=grid,
            in_specs=in_specs,
            out_specs=pl.BlockSpec((RPS, D), lambda i, idx_ref: (i, 0)),
        ),
        out_shape=jax.ShapeDtypeStruct((B_TC, D), jnp.float32),
    )(idx, *([table3] * RPS))


def kernel(X, embed_weight):
    idx = X.reshape(-1)
    table3 = embed_weight.reshape(VOCAB, 1, D)
    out_tc = _tc_gather(idx[:B_TC], table3)
    out_sc = _sc_gather(idx[B_TC:].reshape(NW, NCHUNK, CHUNK), embed_weight)
    out = jnp.concatenate([out_tc, out_sc], axis=0)
    return out.reshape(X.shape[0], X.shape[1], D)


# speedup vs baseline: 2.9331x; 2.9331x over previous
import functools

import jax
import jax.numpy as jnp
from jax import lax
from jax.experimental import pallas as pl
from jax.experimental.pallas import tpu as pltpu
from jax.experimental.pallas import tpu_sc as plsc

VOCAB = 8192
D = 8192
B = 8192

NC = 2
NS = 16
NW = NC * NS
BPW = B // NW
CHUNK = 4
NCHUNK = BPW // CHUNK
NBUF = 3


@jax.jit
def _sc_gather(idx, table):
    mesh = plsc.VectorSubcoreMesh(core_axis_name="c", subcore_axis_name="s")

    @functools.partial(
        pl.kernel,
        out_type=jax.ShapeDtypeStruct((B, D), jnp.float32),
        mesh=mesh,
        scratch_types=[
            pltpu.VMEM((NCHUNK, CHUNK), jnp.int32),
            pltpu.VMEM((NBUF, CHUNK, D), jnp.float32),
            pltpu.SemaphoreType.DMA,
            pltpu.SemaphoreType.DMA,
            pltpu.SemaphoreType.DMA,
            pltpu.SemaphoreType.DMA,
            pltpu.SemaphoreType.DMA,
            pltpu.SemaphoreType.DMA,
        ],
    )
    def k(idx_hbm, table_hbm, out_hbm, idx_v, bufs, g0, g1, g2, s0, s1, s2):
        wid = lax.axis_index("s") * NC + lax.axis_index("c")
        base = wid * BPW
        pltpu.sync_copy(idx_hbm.at[wid], idx_v)

        gsems = (g0, g1, g2)
        ssems = (s0, s1, s2)

        def g_start(c, b):
            pltpu.async_copy(table_hbm.at[idx_v.at[c]], bufs.at[b], gsems[b])

        def g_wait(b):
            pltpu.make_async_copy(
                table_hbm.at[pl.ds(0, CHUNK)], bufs.at[b], gsems[b]
            ).wait()

        def s_start(c, b):
            pltpu.async_copy(
                bufs.at[b], out_hbm.at[pl.ds(base + c * CHUNK, CHUNK)], ssems[b]
            )

        def s_wait(b):
            pltpu.make_async_copy(
                bufs.at[b], out_hbm.at[pl.ds(base, CHUNK)], ssems[b]
            ).wait()

        g_start(0, 0)
        g_start(1, 1)
        for c in range(NCHUNK):
            b = c % NBUF
            g_wait(b)
            s_start(c, b)
            if c + 2 <= NCHUNK - 1:
                bn = (c + 2) % NBUF
                if c != 0:
                    s_wait(bn)
                g_start(c + 2, bn)
        s_wait((NCHUNK - 2) % NBUF)
        s_wait((NCHUNK - 1) % NBUF)
        s_wait((NCHUNK - 3) % NBUF)

    return k(idx, table)


def kernel(X, embed_weight):
    idx = X.reshape(NW, NCHUNK, CHUNK)
    out = _sc_gather(idx, embed_weight)
    return out.reshape(X.shape[0], X.shape[1], D)


# speedup vs baseline: 2.9903x; 1.0195x over previous
import functools

import jax
import jax.numpy as jnp
from jax import lax
from jax.experimental import pallas as pl
from jax.experimental.pallas import tpu as pltpu
from jax.experimental.pallas import tpu_sc as plsc

VOCAB = 8192
D = 8192
B = 8192

NC = 2
NS = 16
NW = NC * NS
BPW = B // NW
CHUNK = 2
NCHUNK = BPW // CHUNK
NBUF = 6
LEAD = 3


@jax.jit
def _sc_gather(idx, table):
    mesh = plsc.VectorSubcoreMesh(core_axis_name="c", subcore_axis_name="s")

    @functools.partial(
        pl.kernel,
        out_type=jax.ShapeDtypeStruct((B, D), jnp.float32),
        mesh=mesh,
        scratch_types=[
            pltpu.VMEM((NCHUNK, CHUNK), jnp.int32),
            pltpu.VMEM((NBUF, CHUNK, D), jnp.float32),
        ]
        + [pltpu.SemaphoreType.DMA] * (2 * NBUF),
    )
    def k(idx_hbm, table_hbm, out_hbm, idx_v, bufs, *sems):
        wid = lax.axis_index("s") * NC + lax.axis_index("c")
        base = wid * BPW
        pltpu.sync_copy(idx_hbm.at[wid], idx_v)

        gsems = sems[:NBUF]
        ssems = sems[NBUF:]

        def g_start(c, b):
            pltpu.async_copy(table_hbm.at[idx_v.at[c]], bufs.at[b], gsems[b])

        def g_wait(b):
            pltpu.make_async_copy(
                table_hbm.at[pl.ds(0, CHUNK)], bufs.at[b], gsems[b]
            ).wait()

        def s_start(c, b):
            pltpu.async_copy(
                bufs.at[b], out_hbm.at[pl.ds(base + c * CHUNK, CHUNK)], ssems[b]
            )

        def s_wait(b):
            pltpu.make_async_copy(
                bufs.at[b], out_hbm.at[pl.ds(base, CHUNK)], ssems[b]
            ).wait()

        def step(c):
            b = c % NBUF
            g_wait(b)
            s_start(c, b)
            cf = c + LEAD
            if cf <= NCHUNK - 1:
                bn = cf % NBUF
                if cf - NBUF >= 0:
                    s_wait(bn)
                g_start(cf, bn)

        for c in range(LEAD):
            g_start(c, c % NBUF)

        for c in range(NBUF):
            step(c)

        def group(j, carry):
            c0 = NBUF * j
            for kk in range(NBUF):
                cc = c0 + kk
                b = kk
                g_wait(b)
                s_start(cc, b)
                bn = (kk + LEAD) % NBUF
                s_wait(bn)
                g_start(cc + LEAD, bn)
            return carry

        NG = (NCHUNK - LEAD) // NBUF
        lax.fori_loop(1, NG, group, 0)

        for c in range(NG * NBUF, NCHUNK):
            step(c)

        for c in range(NCHUNK - NBUF, NCHUNK):
            s_wait(c % NBUF)

    return k(idx, table)


def kernel(X, embed_weight):
    idx = X.reshape(NW, NCHUNK, CHUNK)
    out = _sc_gather(idx, embed_weight)
    return out.reshape(X.shape[0], X.shape[1], D)
